# baseline (speedup 1.0000x reference)
import functools
import jax
import jax.numpy as jnp
from jax import lax
from jax.experimental import pallas as pl
from jax.experimental.pallas import tpu as pltpu
from jax.experimental.pallas import tpu_sc as plsc

_NUM_RAYS = 1024
_TR = 512
_CH = 32


def _sc_li(rows_total):
    info = plsc.get_sparse_core_info()
    nc, ns = info.num_cores, info.num_subcores
    nw = nc * ns
    rows_w = rows_total // nw
    nchunks = rows_w // _CH
    mesh = plsc.VectorSubcoreMesh(core_axis_name="c", subcore_axis_name="s")

    @functools.partial(
        pl.kernel, mesh=mesh,
        out_type=jax.ShapeDtypeStruct((rows_total, _NUM_RAYS), jnp.float32),
        scratch_types=[
            pltpu.VMEM((rows_w,), jnp.int32),
            pltpu.VMEM((_CH, _NUM_RAYS), jnp.float32),
            pltpu.VMEM((_CH, _NUM_RAYS), jnp.float32),
            pltpu.SemaphoreType.DMA,
            pltpu.SemaphoreType.DMA,
            pltpu.SemaphoreType.DMA,
            pltpu.SemaphoreType.DMA,
        ],
    )
    def k(idx_hbm, tblrow_hbm, out_hbm, idx_v, buf0, buf1, g0, g1, s0, s1):
        wid = lax.axis_index("s") * nc + lax.axis_index("c")
        r0 = wid * rows_w
        pltpu.sync_copy(idx_hbm.at[pl.ds(r0, rows_w)], idx_v)
        bufs = (buf0, buf1)
        gsems = (g0, g1)
        ssems = (s0, s1)
        pending = [None, None]
        for ci in range(nchunks):
            p = ci % 2
            if pending[p] is not None:
                pending[p].wait()
            pltpu.async_copy(
                tblrow_hbm.at[idx_v.at[pl.ds(ci * _CH, _CH)]],
                bufs[p], gsems[p]).wait()
            pending[p] = pltpu.async_copy(
                bufs[p], out_hbm.at[pl.ds(r0 + ci * _CH, _CH)], ssems[p])
        for p in range(2):
            if pending[p] is not None:
                pending[p].wait()

    return k


def _tc_body(idx_ref, tbl_ref, out_ld_ref):
    tr = idx_ref.shape[-1]
    nl = tbl_ref.shape[0]
    nb = pl.num_programs(0) // 3
    c = pl.program_id(0) // nb
    idx = idx_ref[0, 0, :]
    onehot = (jax.lax.broadcasted_iota(jnp.int32, (tr, nl), 1) == idx[:, None])
    vals = jax.lax.dot_general(
        onehot.astype(jnp.float32), tbl_ref[...],
        (((1,), (0,)), ((), ())), preferred_element_type=jnp.float32)
    x = vals[:, 0:1]
    y = vals[:, 1:2]
    z = -jnp.abs(vals[:, 2:3])
    n = jnp.maximum(jnp.sqrt(x * x + y * y + z * z), 1e-12)
    col = jnp.where(c == 0, x, jnp.where(c == 1, y, z)) / n
    out_ld_ref[...] = jnp.broadcast_to(col, (tr, _NUM_RAYS))


def kernel(idx, light_direction_xy, light_direction_z, light_intensity):
    b = idx.shape[0]
    nl = light_intensity.shape[0]
    rows_total = 3 * b
    tbl_li = jnp.broadcast_to(jnp.abs(light_intensity), (nl, _NUM_RAYS))
    idx_ext = jnp.concatenate([idx, idx, idx])
    out_li = _sc_li(rows_total)(idx_ext, tbl_li)

    tbl = jnp.concatenate(
        [light_direction_xy, light_direction_z, light_intensity], axis=1)
    nb = b // _TR
    idx3 = idx.reshape(nb, 1, _TR)
    out_ld = pl.pallas_call(
        _tc_body,
        grid=(3 * nb,),
        in_specs=[
            pl.BlockSpec((1, 1, _TR), lambda i, nb=nb: (jax.lax.rem(i, nb), 0, 0)),
            pl.BlockSpec(tbl.shape, lambda i: (0, 0)),
        ],
        out_specs=pl.BlockSpec((_TR, _NUM_RAYS), lambda i: (i, 0)),
        out_shape=jax.ShapeDtypeStruct((rows_total, _NUM_RAYS), jnp.float32),
    )(idx3, tbl)
    out_ld = out_ld.reshape(3, b * _NUM_RAYS).T
    out_li = out_li.reshape(3, b * _NUM_RAYS).T
    return (out_ld, out_li)
